# Initial kernel scaffold; baseline (speedup 1.0000x reference)
import functools

import jax
import jax.numpy as jnp
from jax import lax
from jax.experimental import pallas as pl
from jax.experimental.pallas import tpu as pltpu
from jax.experimental.pallas import tpu_sc as plsc

D = 128
HIST = 32
NCORES = 2
NSUB = 16
NW = NCORES * NSUB
LANES = 16
PB = 8


def _vbcast(vec, i):
    idx = jnp.full((LANES,), i, dtype=jnp.int32)
    return lax.gather(
        vec,
        idx[:, None],
        lax.GatherDimensionNumbers(
            offset_dims=(), collapsed_slice_dims=(0,), start_index_map=(0,)
        ),
        (1,),
        mode=lax.GatherScatterMode.PROMISE_IN_BOUNDS,
    )


def _proj_body(item_blk, wagg, rat, bagg, out_blk, ratq_out):
    out_blk[:, :] = jnp.dot(
        item_blk[:, :], wagg[:D, :], preferred_element_type=jnp.float32
    )

    @pl.when(pl.program_id(0) == 0)
    def _():
        ratq_out[:, :] = (
            jnp.dot(rat[:, :], wagg[D:, :], preferred_element_type=jnp.float32)
            + bagg[:, :]
        )


def _project_tables(item_table, rating_table, W_agg, b_agg):
    n = item_table.shape[0]
    blk = 1000
    assert n % blk == 0
    rat8 = jnp.pad(rating_table, ((0, 8 - rating_table.shape[0]), (0, 0)))
    return pl.pallas_call(
        _proj_body,
        grid=(n // blk,),
        in_specs=[
            pl.BlockSpec((blk, D), lambda i: (i, 0)),
            pl.BlockSpec((2 * D, D), lambda i: (0, 0)),
            pl.BlockSpec((8, D), lambda i: (0, 0)),
            pl.BlockSpec((1, D), lambda i: (0, 0)),
        ],
        out_specs=[
            pl.BlockSpec((blk, D), lambda i: (i, 0)),
            pl.BlockSpec((8, D), lambda i: (0, 0)),
        ],
        out_shape=[
            jax.ShapeDtypeStruct((n, D), jnp.float32),
            jax.ShapeDtypeStruct((8, D), jnp.float32),
        ],
    )(item_table, W_agg, rat8, b_agg.reshape(1, D))


def _sc_body(
    nodes_h, hidx_h, hrat_h, iproj_h, ratq_h, feat_h,
    neigh_out, self_out,
    nodes_v, hidx_v, hrat_v, rows_v, feat_v, ratq_v, neigh_v,
    sem_a, sem_b,
):
    batch = nodes_h.shape[0]
    per_w = batch // NW
    wid = lax.axis_index("s") * NCORES + lax.axis_index("c")
    base = wid * per_w

    pltpu.sync_copy(ratq_h, ratq_v)
    inv = jnp.float32(1.0 / HIST)

    def chunk(ch, carry):
        cb = base + ch * PB
        pltpu.sync_copy(nodes_h.at[pl.ds(cb, PB)], nodes_v)
        cp1 = pltpu.async_copy(hidx_h.at[nodes_v], hidx_v, sem_a)
        cp2 = pltpu.async_copy(hrat_h.at[nodes_v], hrat_v, sem_a)
        cp3 = pltpu.async_copy(feat_h.at[nodes_v], feat_v, sem_a)
        cp1.wait()
        cp2.wait()
        cp3.wait()
        pltpu.sync_copy(feat_v, self_out.at[pl.ds(cb, PB)])
        descs = [
            pltpu.async_copy(iproj_h.at[hidx_v.at[b]], rows_v.at[b], sem_b)
            for b in range(PB)
        ]
        for d in descs:
            d.wait()
        for b in range(PB):
            rat0 = hrat_v[b, pl.ds(0, LANES)]
            rat1 = hrat_v[b, pl.ds(LANES, LANES)]
            cols = [
                lax.iota(jnp.int32, LANES) + c * LANES for c in range(D // LANES)
            ]

            def lstep(i, accs):
                r0 = _vbcast(rat0, i)
                r1 = _vbcast(rat1, i)
                out = []
                for c in range(D // LANES):
                    q0 = plsc.load_gather(ratq_v, [r0, cols[c]])
                    q1 = plsc.load_gather(ratq_v, [r1, cols[c]])
                    it0 = rows_v[b, i, pl.ds(c * LANES, LANES)]
                    it1 = rows_v[b, i + LANES, pl.ds(c * LANES, LANES)]
                    out.append(
                        accs[c]
                        + jnp.maximum(it0 + q0, 0.0)
                        + jnp.maximum(it1 + q1, 0.0)
                    )
                return tuple(out)

            accs = lax.fori_loop(
                0,
                LANES,
                lstep,
                tuple(jnp.zeros((LANES,), jnp.float32) for _ in range(D // LANES)),
            )
            for c in range(D // LANES):
                neigh_v[b, pl.ds(c * LANES, LANES)] = accs[c] * inv
        pltpu.sync_copy(neigh_v, neigh_out.at[pl.ds(cb, PB)])
        return carry

    lax.fori_loop(0, per_w // PB, chunk, 0)


def _sc_aggregate(nodes, hist_idx, hist_rating, item_proj, ratq, feature_table):
    batch = nodes.shape[0]
    mesh = plsc.VectorSubcoreMesh(core_axis_name="c", subcore_axis_name="s")
    fn = pl.kernel(
        _sc_body,
        out_type=[
            jax.ShapeDtypeStruct((batch, D), jnp.float32),
            jax.ShapeDtypeStruct((batch, D), jnp.float32),
        ],
        mesh=mesh,
        scratch_types=[
            pltpu.VMEM((PB,), jnp.int32),
            pltpu.VMEM((PB, HIST), jnp.int32),
            pltpu.VMEM((PB, HIST), jnp.int32),
            pltpu.VMEM((PB, HIST, D), jnp.float32),
            pltpu.VMEM((PB, D), jnp.float32),
            pltpu.VMEM((8, D), jnp.float32),
            pltpu.VMEM((PB, D), jnp.float32),
            pltpu.SemaphoreType.DMA,
            pltpu.SemaphoreType.DMA,
        ],
    )
    return fn(nodes, hist_idx, hist_rating, item_proj, ratq, feature_table)


def _final_body(s_blk, n_blk, w1, b1r, o_blk):
    o = (
        jnp.dot(s_blk[:, :], w1[:D, :], preferred_element_type=jnp.float32)
        + jnp.dot(n_blk[:, :], w1[D:, :], preferred_element_type=jnp.float32)
        + b1r[:, :]
    )
    o_blk[:, :] = jnp.maximum(o, 0.0)


def _final(selfF, neigh, W1, b1):
    batch = selfF.shape[0]
    blk = 2048
    return pl.pallas_call(
        _final_body,
        grid=(batch // blk,),
        in_specs=[
            pl.BlockSpec((blk, D), lambda i: (i, 0)),
            pl.BlockSpec((blk, D), lambda i: (i, 0)),
            pl.BlockSpec((2 * D, D), lambda i: (0, 0)),
            pl.BlockSpec((1, D), lambda i: (0, 0)),
        ],
        out_specs=pl.BlockSpec((blk, D), lambda i: (i, 0)),
        out_shape=jax.ShapeDtypeStruct((batch, D), jnp.float32),
    )(selfF, neigh, W1, b1.reshape(1, D))


def kernel(nodes, hist_idx, hist_rating, feature_table, item_table,
           rating_table, W_agg, b_agg, W1, b1):
    nodes = nodes.astype(jnp.int32)
    hist_idx = hist_idx.astype(jnp.int32)
    hist_rating = hist_rating.astype(jnp.int32)
    item_proj, ratq = _project_tables(item_table, rating_table, W_agg, b_agg)
    neigh, selfF = _sc_aggregate(
        nodes, hist_idx, hist_rating, item_proj, ratq, feature_table
    )
    return _final(selfF, neigh, W1, b1)


# speedup vs baseline: 7.4670x; 7.4670x over previous
import functools

import jax
import jax.numpy as jnp
from jax import lax
from jax.experimental import pallas as pl
from jax.experimental.pallas import tpu as pltpu
from jax.experimental.pallas import tpu_sc as plsc

D = 128
HIST = 32
NCORES = 2
NSUB = 16
NW = NCORES * NSUB
LANES = 16
PB = 8


def _vbcast(vec, i):
    idx = jnp.full((LANES,), i, dtype=jnp.int32)
    return lax.gather(
        vec,
        idx[:, None],
        lax.GatherDimensionNumbers(
            offset_dims=(), collapsed_slice_dims=(0,), start_index_map=(0,)
        ),
        (1,),
        mode=lax.GatherScatterMode.PROMISE_IN_BOUNDS,
    )


def _proj_body(item_blk, wagg, rat, bagg, out_blk, ratq_out):
    out_blk[:, :] = jnp.dot(
        item_blk[:, :], wagg[:D, :], preferred_element_type=jnp.float32
    )

    @pl.when(pl.program_id(0) == 0)
    def _():
        ratq_out[:, :] = (
            jnp.dot(rat[:, :], wagg[D:, :], preferred_element_type=jnp.float32)
            + bagg[:, :]
        )


def _project_tables(item_table, rating_table, W_agg, b_agg):
    n = item_table.shape[0]
    blk = 1000
    assert n % blk == 0
    rat8 = jnp.pad(rating_table, ((0, 8 - rating_table.shape[0]), (0, 0)))
    return pl.pallas_call(
        _proj_body,
        grid=(n // blk,),
        in_specs=[
            pl.BlockSpec((blk, D), lambda i: (i, 0)),
            pl.BlockSpec((2 * D, D), lambda i: (0, 0)),
            pl.BlockSpec((8, D), lambda i: (0, 0)),
            pl.BlockSpec((1, D), lambda i: (0, 0)),
        ],
        out_specs=[
            pl.BlockSpec((blk, D), lambda i: (i, 0)),
            pl.BlockSpec((8, D), lambda i: (0, 0)),
        ],
        out_shape=[
            jax.ShapeDtypeStruct((n, D), jnp.float32),
            jax.ShapeDtypeStruct((8, D), jnp.float32),
        ],
    )(item_table, W_agg, rat8, b_agg.reshape(1, D))


def _sc_body(
    nodes_h, hidx_h, hrat_h, iproj_h, ratq_h, feat_h,
    neigh_out, self_out,
    nodes_v, hidx_v, hrat_v, rows_v, feat_v, ratq_v, neigh_v,
    sem_a, sem_b,
):
    batch = nodes_h.shape[0]
    per_w = batch // NW
    wid = lax.axis_index("s") * NCORES + lax.axis_index("c")
    base = wid * per_w

    pltpu.sync_copy(ratq_h, ratq_v)
    inv = jnp.float32(1.0 / HIST)

    def chunk(ch, carry):
        cb = base + ch * PB
        pltpu.sync_copy(nodes_h.at[pl.ds(cb, PB)], nodes_v)
        cp1 = pltpu.async_copy(hidx_h.at[nodes_v], hidx_v, sem_a)
        cp2 = pltpu.async_copy(hrat_h.at[nodes_v], hrat_v, sem_a)
        cp3 = pltpu.async_copy(feat_h.at[nodes_v], feat_v, sem_a)
        cp1.wait()
        cp2.wait()
        cp3.wait()
        pltpu.sync_copy(feat_v, self_out.at[pl.ds(cb, PB)])
        descs = [
            pltpu.async_copy(iproj_h.at[hidx_v.at[b]], rows_v.at[b], sem_b)
            for b in range(PB)
        ]
        for d in descs:
            d.wait()
        for b in range(PB):
            rat0 = hrat_v[b, pl.ds(0, LANES)]
            rat1 = hrat_v[b, pl.ds(LANES, LANES)]
            cols = [
                lax.iota(jnp.int32, LANES) + c * LANES for c in range(D // LANES)
            ]

            def lstep(i, accs):
                r0 = _vbcast(rat0, i) * D
                r1 = _vbcast(rat1, i) * D
                out = []
                for c in range(D // LANES):
                    q0 = plsc.load_gather(ratq_v, [r0 + cols[c]])
                    q1 = plsc.load_gather(ratq_v, [r1 + cols[c]])
                    it0 = rows_v[b, i, pl.ds(c * LANES, LANES)]
                    it1 = rows_v[b, i + LANES, pl.ds(c * LANES, LANES)]
                    out.append(
                        accs[c]
                        + jnp.maximum(it0 + q0, 0.0)
                        + jnp.maximum(it1 + q1, 0.0)
                    )
                return tuple(out)

            accs = lax.fori_loop(
                0,
                LANES,
                lstep,
                tuple(jnp.zeros((LANES,), jnp.float32) for _ in range(D // LANES)),
            )
            for c in range(D // LANES):
                neigh_v[b, pl.ds(c * LANES, LANES)] = accs[c] * inv
        pltpu.sync_copy(neigh_v, neigh_out.at[pl.ds(cb, PB)])
        return carry

    lax.fori_loop(0, per_w // PB, chunk, 0)


def _sc_aggregate(nodes, hist_idx, hist_rating, item_proj, ratq, feature_table):
    batch = nodes.shape[0]
    mesh = plsc.VectorSubcoreMesh(
        core_axis_name="c", subcore_axis_name="s",
        num_cores=NCORES, num_subcores=NSUB,
    )
    fn = pl.kernel(
        _sc_body,
        out_type=[
            jax.ShapeDtypeStruct((batch, D), jnp.float32),
            jax.ShapeDtypeStruct((batch, D), jnp.float32),
        ],
        mesh=mesh,
        scratch_types=[
            pltpu.VMEM((PB,), jnp.int32),
            pltpu.VMEM((PB, HIST), jnp.int32),
            pltpu.VMEM((PB, HIST), jnp.int32),
            pltpu.VMEM((PB, HIST, D), jnp.float32),
            pltpu.VMEM((PB, D), jnp.float32),
            pltpu.VMEM((8 * D,), jnp.float32),
            pltpu.VMEM((PB, D), jnp.float32),
            pltpu.SemaphoreType.DMA,
            pltpu.SemaphoreType.DMA,
        ],
        compiler_params=pltpu.CompilerParams(
            needs_layout_passes=False, use_tc_tiling_on_sc=False
        ),
    )
    return fn(nodes, hist_idx, hist_rating, item_proj, ratq.reshape(-1),
              feature_table)


def _final_body(s_blk, n_blk, w1, b1r, o_blk):
    o = (
        jnp.dot(s_blk[:, :], w1[:D, :], preferred_element_type=jnp.float32)
        + jnp.dot(n_blk[:, :], w1[D:, :], preferred_element_type=jnp.float32)
        + b1r[:, :]
    )
    o_blk[:, :] = jnp.maximum(o, 0.0)


def _final(selfF, neigh, W1, b1):
    batch = selfF.shape[0]
    blk = 2048
    return pl.pallas_call(
        _final_body,
        grid=(batch // blk,),
        in_specs=[
            pl.BlockSpec((blk, D), lambda i: (i, 0)),
            pl.BlockSpec((blk, D), lambda i: (i, 0)),
            pl.BlockSpec((2 * D, D), lambda i: (0, 0)),
            pl.BlockSpec((1, D), lambda i: (0, 0)),
        ],
        out_specs=pl.BlockSpec((blk, D), lambda i: (i, 0)),
        out_shape=jax.ShapeDtypeStruct((batch, D), jnp.float32),
    )(selfF, neigh, W1, b1.reshape(1, D))


def kernel(nodes, hist_idx, hist_rating, feature_table, item_table,
           rating_table, W_agg, b_agg, W1, b1):
    nodes = nodes.astype(jnp.int32)
    hist_idx = hist_idx.astype(jnp.int32)
    hist_rating = hist_rating.astype(jnp.int32)
    item_proj, ratq = _project_tables(item_table, rating_table, W_agg, b_agg)
    neigh, selfF = _sc_aggregate(
        nodes, hist_idx, hist_rating, item_proj, ratq, feature_table
    )
    return _final(selfF, neigh, W1, b1)


# speedup vs baseline: 11.1798x; 1.4972x over previous
import functools

import jax
import jax.numpy as jnp
from jax import lax
from jax.experimental import pallas as pl
from jax.experimental.pallas import tpu as pltpu
from jax.experimental.pallas import tpu_sc as plsc

D = 128
HIST = 32
NCORES = 2
NSUB = 16
NW = NCORES * NSUB
LANES = 16
PB = 8


def _vbcast(vec, i):
    idx = jnp.full((LANES,), i, dtype=jnp.int32)
    return lax.gather(
        vec,
        idx[:, None],
        lax.GatherDimensionNumbers(
            offset_dims=(), collapsed_slice_dims=(0,), start_index_map=(0,)
        ),
        (1,),
        mode=lax.GatherScatterMode.PROMISE_IN_BOUNDS,
    )


def _proj_body(item_blk, wagg, rat, bagg, out_blk, ratq_out):
    out_blk[:, :] = jnp.dot(
        item_blk[:, :], wagg[:D, :], preferred_element_type=jnp.float32
    )

    @pl.when(pl.program_id(0) == 0)
    def _():
        ratq_out[:, :] = (
            jnp.dot(rat[:, :], wagg[D:, :], preferred_element_type=jnp.float32)
            + bagg[:, :]
        )


def _project_tables(item_table, rating_table, W_agg, b_agg):
    n = item_table.shape[0]
    blk = 1000
    assert n % blk == 0
    rat8 = jnp.pad(rating_table, ((0, 8 - rating_table.shape[0]), (0, 0)))
    return pl.pallas_call(
        _proj_body,
        grid=(n // blk,),
        in_specs=[
            pl.BlockSpec((blk, D), lambda i: (i, 0)),
            pl.BlockSpec((2 * D, D), lambda i: (0, 0)),
            pl.BlockSpec((8, D), lambda i: (0, 0)),
            pl.BlockSpec((1, D), lambda i: (0, 0)),
        ],
        out_specs=[
            pl.BlockSpec((blk, D), lambda i: (i, 0)),
            pl.BlockSpec((8, D), lambda i: (0, 0)),
        ],
        out_shape=[
            jax.ShapeDtypeStruct((n, D), jnp.float32),
            jax.ShapeDtypeStruct((8, D), jnp.float32),
        ],
    )(item_table, W_agg, rat8, b_agg.reshape(1, D))


def _sc_body(
    nodes_h, hidx_h, hrat_h, iproj_h, ratq_h, feat_h,
    neigh_out, self_out,
    nodes_v, hidx_v, hrat_v, rows_v, feat_v, ratq_v, neigh_v,
    sem_a, sem_b, sem_s,
):
    batch = nodes_h.shape[0]
    per_w = batch // NW
    nch = per_w // PB
    wid = lax.axis_index("s") * NCORES + lax.axis_index("c")
    base = wid * per_w
    inv = jnp.float32(1.0 / HIST)

    pltpu.sync_copy(ratq_h, ratq_v)
    pltpu.sync_copy(nodes_h.at[pl.ds(base, per_w)], nodes_v)
    hist_cps = []
    for k in range(per_w // 128):
        sl = pl.ds(k * 128, 128)
        hist_cps.append(
            pltpu.async_copy(hidx_h.at[nodes_v.at[sl]], hidx_v.at[sl], sem_a)
        )
        hist_cps.append(
            pltpu.async_copy(hrat_h.at[nodes_v.at[sl]], hrat_v.at[sl], sem_a)
        )
    for cp in hist_cps:
        cp.wait()

    def issue_chunk(ch, buf):
        cb = ch * PB
        for b in range(PB):
            pltpu.async_copy(iproj_h.at[hidx_v.at[cb + b]], rows_v.at[buf, b],
                             sem_b)
        pltpu.async_copy(feat_h.at[nodes_v.at[pl.ds(cb, PB)]], feat_v.at[buf],
                         sem_a)

    def wait_chunk(buf):
        for b in range(PB):
            pltpu.make_async_copy(iproj_h.at[hidx_v.at[b]], rows_v.at[buf, b],
                                  sem_b).wait()
        pltpu.make_async_copy(feat_h.at[nodes_v.at[pl.ds(0, PB)]],
                              feat_v.at[buf], sem_a).wait()

    def wait_stores():
        pltpu.make_async_copy(neigh_v.at[0], neigh_out.at[pl.ds(0, PB)],
                              sem_s).wait()
        pltpu.make_async_copy(neigh_v.at[0], neigh_out.at[pl.ds(0, PB)],
                              sem_s).wait()

    issue_chunk(0, 0)

    cols = [lax.iota(jnp.int32, LANES) + c * LANES for c in range(D // LANES)]

    @pl.loop(0, nch, step=2)
    def _outer(c0):
        for par in range(2):
            cc = c0 + par
            cb = cc * PB

            @pl.when(cc >= 1)
            def _():
                wait_stores()

            @pl.when(cc + 1 < nch)
            def _():
                issue_chunk(cc + 1, 1 - par)

            wait_chunk(par)

            for b in range(PB):
                rat0 = hrat_v[cb + b, pl.ds(0, LANES)]
                rat1 = hrat_v[cb + b, pl.ds(LANES, LANES)]

                def lstep(i, accs):
                    r0 = _vbcast(rat0, i) * D
                    r1 = _vbcast(rat1, i) * D
                    out = []
                    for c in range(D // LANES):
                        q0 = plsc.load_gather(ratq_v, [r0 + cols[c]])
                        q1 = plsc.load_gather(ratq_v, [r1 + cols[c]])
                        it0 = rows_v[par, b, i, pl.ds(c * LANES, LANES)]
                        it1 = rows_v[par, b, i + LANES, pl.ds(c * LANES, LANES)]
                        out.append(
                            accs[c]
                            + jnp.maximum(it0 + q0, 0.0)
                            + jnp.maximum(it1 + q1, 0.0)
                        )
                    return tuple(out)

                accs = lax.fori_loop(
                    0,
                    LANES,
                    lstep,
                    tuple(jnp.zeros((LANES,), jnp.float32)
                          for _ in range(D // LANES)),
                )
                for c in range(D // LANES):
                    neigh_v[par, b, pl.ds(c * LANES, LANES)] = accs[c] * inv

            pltpu.async_copy(neigh_v.at[par], neigh_out.at[pl.ds(base + cb, PB)],
                             sem_s)
            pltpu.async_copy(feat_v.at[par], self_out.at[pl.ds(base + cb, PB)],
                             sem_s)

    wait_stores()


def _sc_aggregate(nodes, hist_idx, hist_rating, item_proj, ratq, feature_table):
    batch = nodes.shape[0]
    mesh = plsc.VectorSubcoreMesh(
        core_axis_name="c", subcore_axis_name="s",
        num_cores=NCORES, num_subcores=NSUB,
    )
    fn = pl.kernel(
        _sc_body,
        out_type=[
            jax.ShapeDtypeStruct((batch, D), jnp.float32),
            jax.ShapeDtypeStruct((batch, D), jnp.float32),
        ],
        mesh=mesh,
        scratch_types=[
            pltpu.VMEM((batch // NW,), jnp.int32),
            pltpu.VMEM((batch // NW, HIST), jnp.int32),
            pltpu.VMEM((batch // NW, HIST), jnp.int32),
            pltpu.VMEM((2, PB, HIST, D), jnp.float32),
            pltpu.VMEM((2, PB, D), jnp.float32),
            pltpu.VMEM((8 * D,), jnp.float32),
            pltpu.VMEM((2, PB, D), jnp.float32),
            pltpu.SemaphoreType.DMA,
            pltpu.SemaphoreType.DMA,
            pltpu.SemaphoreType.DMA,
        ],
        compiler_params=pltpu.CompilerParams(
            needs_layout_passes=False, use_tc_tiling_on_sc=False
        ),
    )
    return fn(nodes, hist_idx, hist_rating, item_proj, ratq.reshape(-1),
              feature_table)


def _final_body(s_blk, n_blk, w1, b1r, o_blk):
    o = (
        jnp.dot(s_blk[:, :], w1[:D, :], preferred_element_type=jnp.float32)
        + jnp.dot(n_blk[:, :], w1[D:, :], preferred_element_type=jnp.float32)
        + b1r[:, :]
    )
    o_blk[:, :] = jnp.maximum(o, 0.0)


def _final(selfF, neigh, W1, b1):
    batch = selfF.shape[0]
    blk = 2048
    return pl.pallas_call(
        _final_body,
        grid=(batch // blk,),
        in_specs=[
            pl.BlockSpec((blk, D), lambda i: (i, 0)),
            pl.BlockSpec((blk, D), lambda i: (i, 0)),
            pl.BlockSpec((2 * D, D), lambda i: (0, 0)),
            pl.BlockSpec((1, D), lambda i: (0, 0)),
        ],
        out_specs=pl.BlockSpec((blk, D), lambda i: (i, 0)),
        out_shape=jax.ShapeDtypeStruct((batch, D), jnp.float32),
    )(selfF, neigh, W1, b1.reshape(1, D))


def kernel(nodes, hist_idx, hist_rating, feature_table, item_table,
           rating_table, W_agg, b_agg, W1, b1):
    nodes = nodes.astype(jnp.int32)
    hist_idx = hist_idx.astype(jnp.int32)
    hist_rating = hist_rating.astype(jnp.int32)
    item_proj, ratq = _project_tables(item_table, rating_table, W_agg, b_agg)
    neigh, selfF = _sc_aggregate(
        nodes, hist_idx, hist_rating, item_proj, ratq, feature_table
    )
    return _final(selfF, neigh, W1, b1)
